# baseline (speedup 1.0000x reference)
import functools

import jax
import jax.numpy as jnp
from jax import lax
from jax.experimental import pallas as pl
from jax.experimental.pallas import tpu as pltpu
from jax.experimental.pallas import tpu_sc as plsc

N = 10000
E = 320000
TWO_E = 2 * E
H = 128
SCH = 1024
NSC = TWO_E // SCH
TE = 1024

_BASE = 624


def _tile_start(s):
    return 8 * (78 * s + jnp.minimum(s, 2))


def _mesh():
    return plsc.VectorSubcoreMesh(
        core_axis_name="c", subcore_axis_name="s",
        num_cores=2, num_subcores=16)

_F32 = jnp.float32


def _mm(x, w):
    return lax.dot_general(x, w, (((1,), (0,)), ((), ())),
                           preferred_element_type=_F32)


def _mmT(x, y):
    return lax.dot_general(x, y, (((0,), (0,)), ((), ())),
                           preferred_element_type=_F32)


def _sc_main(nid2, pnid2, er, nr):
    out_type = (
        jax.ShapeDtypeStruct((TWO_E, H), _F32),
        jax.ShapeDtypeStruct((N, H), _F32),
        jax.ShapeDtypeStruct((N, H), _F32),
        jax.ShapeDtypeStruct((N, H), _F32),
    )
    scratch = [
        pltpu.VMEM((8, 128), jnp.int32),
        pltpu.VMEM((8, 128), jnp.int32),
        pltpu.VMEM((256, H), _F32),
        pltpu.VMEM_SHARED((N, H), _F32),
        pltpu.SemaphoreType.DMA,
    ]

    @functools.partial(pl.kernel, out_type=out_type, mesh=_mesh(),
                       scratch_types=scratch)
    def k(nid_h, pnid_h, er_h, nr_h, p_h, es_h, ns_h, deg_h,
          idx_v, pidx_v, buf_v, acc_sh, sem):
        c = lax.axis_index("c")
        s = lax.axis_index("s")
        zeros16 = jnp.zeros((16,), _F32)
        ones16 = jnp.ones((16,), _F32)
        start = _tile_start(s)
        nck = jnp.where(s < 1, NSC // 16 + 1, NSC // 16)

        def fill_buf(val16, nrows):
            def frow(r, _):
                for l in range(H // 16):
                    buf_v[r, pl.ds(l * 16, 16)] = val16
                return 0
            lax.fori_loop(0, nrows, frow, 0)

        def zero_acc():
            pltpu.sync_copy(buf_v, acc_sh.at[pl.ds(start, 256)])
            pltpu.sync_copy(buf_v, acc_sh.at[pl.ds(start + 256, 256)])
            pltpu.sync_copy(buf_v.at[pl.ds(0, _BASE - 512)],
                            acc_sh.at[pl.ds(start + 512, _BASE - 512)])

            @pl.when(s < 2)
            def _():
                pltpu.sync_copy(buf_v.at[pl.ds(0, 8)],
                                acc_sh.at[pl.ds(start + _BASE, 8)])

        def copy_acc_out(dst_h):
            pltpu.sync_copy(acc_sh.at[pl.ds(start, _BASE)],
                            dst_h.at[pl.ds(start, _BASE)])

            @pl.when(s < 2)
            def _():
                pltpu.sync_copy(acc_sh.at[pl.ds(start + _BASE, 8)],
                                dst_h.at[pl.ds(start + _BASE, 8)])

        fill_buf(zeros16, 256)
        zero_acc()

        @pl.when(c == 0)
        def _():
            plsc.subcore_barrier()
            fill_buf(ones16, 128)

            def dchunk(kk, _):
                sc = s + kk * 16
                pltpu.sync_copy(nid_h.at[pl.ds(sc * 8, 8)], idx_v)
                for j in range(8):
                    pltpu.sync_copy(buf_v.at[pl.ds(0, 128)],
                                    acc_sh.at[idx_v.at[j]], add=True)
                return 0

            lax.fori_loop(0, nck, dchunk, 0)
            plsc.subcore_barrier()
            copy_acc_out(deg_h)
            fill_buf(zeros16, 128)
            plsc.subcore_barrier()
            zero_acc()

        plsc.subcore_barrier()

        def chunk(kk, _):
            sc = s + kk * 16
            g0 = sc * 8
            pltpu.sync_copy(nid_h.at[pl.ds(g0, 8)], idx_v)

            @pl.when(c == 0)
            def _():
                for q in range(4):
                    row0 = sc * SCH + q * 256
                    pltpu.sync_copy(er_h.at[pl.ds(row0, 256)], buf_v)
                    for j in range(2):
                        pltpu.sync_copy(
                            buf_v.at[pl.ds(j * 128, 128)],
                            acc_sh.at[idx_v.at[q * 2 + j]], add=True)

            @pl.when(c == 1)
            def _():
                pltpu.sync_copy(pnid_h.at[pl.ds(g0, 8)], pidx_v)
                for q in range(4):
                    row0 = sc * SCH + q * 256
                    d0 = pltpu.async_copy(nr_h.at[pidx_v.at[q * 2]],
                                          buf_v.at[pl.ds(0, 128)], sem)
                    d1 = pltpu.async_copy(nr_h.at[pidx_v.at[q * 2 + 1]],
                                          buf_v.at[pl.ds(128, 128)], sem)
                    d0.wait()
                    d1.wait()
                    pltpu.sync_copy(buf_v, p_h.at[pl.ds(row0, 256)])
                    for j in range(2):
                        pltpu.sync_copy(
                            buf_v.at[pl.ds(j * 128, 128)],
                            acc_sh.at[idx_v.at[q * 2 + j]], add=True)
            return 0

        lax.fori_loop(0, nck, chunk, 0)
        plsc.subcore_barrier()

        @pl.when(c == 0)
        def _():
            copy_acc_out(es_h)

        @pl.when(c == 1)
        def _():
            copy_acc_out(ns_h)

    return k(nid2, pnid2, er, nr)


def _sc_aux(nid2, s_tbl):
    out_type = jax.ShapeDtypeStruct((TWO_E, H), _F32)
    scratch = [
        pltpu.VMEM((8, 128), jnp.int32),
        pltpu.VMEM((256, H), _F32),
        pltpu.SemaphoreType.DMA,
    ]

    @functools.partial(pl.kernel, out_type=out_type, mesh=_mesh(),
                       scratch_types=scratch)
    def k(nid_h, s_h, sa_h, idx_v, ebuf_v, sem):
        c = lax.axis_index("c")
        s = lax.axis_index("s")
        wid = c * 16 + s

        nck = jnp.where(wid < NSC - 32 * (NSC // 32), NSC // 32 + 1,
                        NSC // 32)

        def chunk(kk, _):
            sc = wid + kk * 32
            g0 = sc * 8
            pltpu.sync_copy(nid_h.at[pl.ds(g0, 8)], idx_v)
            for q in range(4):
                row0 = sc * SCH + q * 256
                d0 = pltpu.async_copy(s_h.at[idx_v.at[q * 2]],
                                      ebuf_v.at[pl.ds(0, 128)], sem)
                d1 = pltpu.async_copy(s_h.at[idx_v.at[q * 2 + 1]],
                                      ebuf_v.at[pl.ds(128, 128)], sem)
                d0.wait()
                d1.wait()
                pltpu.sync_copy(ebuf_v, sa_h.at[pl.ds(row0, 256)])
            return 0

        lax.fori_loop(0, nck, chunk, 0)

    return k(nid2, s_tbl)


def _ks_body(nr_ref, degn_ref, s_ref):
    s_ref[...] = degn_ref[...] * nr_ref[...]


def _ks(nr, degn):
    return pl.pallas_call(
        _ks_body,
        out_shape=jax.ShapeDtypeStruct((N, H), _F32),
    )(nr, degn)


def _gram_body(x_ref, y_ref, out_ref):
    i = pl.program_id(0)
    g = _mmT(x_ref[...], y_ref[...])

    @pl.when(i == 0)
    def _():
        out_ref[...] = g

    @pl.when(i != 0)
    def _():
        out_ref[...] += g


def _gram(x, y):
    return pl.pallas_call(
        _gram_body,
        grid=(TWO_E // TE,),
        in_specs=[pl.BlockSpec((TE, H), lambda i: (i, 0)),
                  pl.BlockSpec((TE, H), lambda i: (i, 0))],
        out_specs=pl.BlockSpec((H, H), lambda i: (0, 0)),
        out_shape=jax.ShapeDtypeStruct((H, H), _F32),
    )(x, y)


def _k1_body(c11_ref, c13_ref, es_ref, ns_ref, nr_ref, degn_ref,
             wa_ref, wb_ref, wc_ref, g1_ref, b1_ref,
             scale1_ref, shift1_ref):
    deg = degn_ref[...]
    nr = nr_ref[...]
    es = es_ref[...]
    ns = ns_ref[...]
    d1nr = deg * nr
    d2nr = deg * d1nr
    n = jnp.float32(TWO_E)

    wa = wa_ref[...]
    wb = wb_ref[...]
    wc = wc_ref[...]

    ey_es = jnp.sum(es, axis=0, keepdims=True)
    ey_d2 = jnp.sum(d2nr, axis=0, keepdims=True)
    ey_d1 = jnp.sum(d1nr, axis=0, keepdims=True)
    mu1 = (_mm(ey_es, wa) + _mm(ey_d2, wb) + _mm(ey_d1, wc)) / n

    u11 = c11_ref[...]
    u13 = c13_ref[...]
    u12 = _mmT(deg * es, nr)
    u22 = _mmT(deg * d2nr, nr)
    u23 = _mmT(d1nr, ns)
    u33 = _mmT(d1nr, nr)

    mt = _mm(u11, wa) + _mm(u12, wb) + _mm(u13, wc)
    mm_ = _mmT(u12, wa) + _mm(u22, wb) + _mm(u23, wc)
    mb = _mmT(u13, wa) + _mmT(u23, wb) + _mm(u33, wc)
    eh2 = (jnp.sum(wa * mt, axis=0, keepdims=True)
           + jnp.sum(wb * mm_, axis=0, keepdims=True)
           + jnp.sum(wc * mb, axis=0, keepdims=True)) / n

    var1 = eh2 - mu1 * mu1
    isig = lax.rsqrt(var1 + 1e-5)
    g1 = g1_ref[...]
    scale1_ref[...] = isig * g1
    shift1_ref[...] = b1_ref[...] - mu1 * isig * g1


def _k1(c11, c13, es, ns, nr, degn, wa, wb, wc, g1, b1):
    return pl.pallas_call(
        _k1_body,
        out_shape=[jax.ShapeDtypeStruct((1, 2 * H), _F32),
                   jax.ShapeDtypeStruct((1, 2 * H), _F32)],
    )(c11, c13, es, ns, nr, degn, wa, wb, wc, g1, b1)


def _k2_body(er_ref, sa_ref, p_ref, sc1_ref, sh1_ref,
             wa_ref, wb_ref, wc_ref, w2_ref,
             h2_ref, s2_ref, q2_ref):
    i = pl.program_id(0)
    h1 = (_mm(er_ref[...], wa_ref[...]) + _mm(sa_ref[...], wb_ref[...])
          + _mm(p_ref[...], wc_ref[...]))
    x = jnp.maximum(h1 * sc1_ref[...] + sh1_ref[...], 0.0)
    h2 = _mm(x, w2_ref[...])
    h2_ref[...] = h2

    se = jnp.sum(h2, axis=0, keepdims=True)
    qe = jnp.sum(h2 * h2, axis=0, keepdims=True)

    @pl.when(i == 0)
    def _():
        s2_ref[...] = se
        q2_ref[...] = qe

    @pl.when(i != 0)
    def _():
        s2_ref[...] += se
        q2_ref[...] += qe


def _k2(er, sa, p, scale1, shift1, wa, wb, wc, w2):
    full = lambda r, c: pl.BlockSpec((r, c), lambda i: (0, 0))
    row = lambda: pl.BlockSpec((TE, H), lambda i: (i, 0))
    return pl.pallas_call(
        _k2_body,
        grid=(TWO_E // TE,),
        in_specs=[row(), row(), row(),
                  full(1, 2 * H), full(1, 2 * H),
                  full(H, 2 * H), full(H, 2 * H), full(H, 2 * H),
                  full(2 * H, H)],
        out_specs=[pl.BlockSpec((TE, H), lambda i: (i, 0)),
                   pl.BlockSpec((1, H), lambda i: (0, 0)),
                   pl.BlockSpec((1, H), lambda i: (0, 0))],
        out_shape=[jax.ShapeDtypeStruct((TWO_E, H), _F32),
                   jax.ShapeDtypeStruct((1, H), _F32),
                   jax.ShapeDtypeStruct((1, H), _F32)],
    )(er, sa, p, scale1, shift1, wa, wb, wc, w2)


def _k3_body(h2_ref, s2_ref, q2_ref, g2_ref, b2_ref, out_ref):
    n = jnp.float32(TWO_E)
    mu = s2_ref[...] / n
    var = q2_ref[...] / n - mu * mu
    sc = lax.rsqrt(var + 1e-5) * g2_ref[...]
    sh = b2_ref[...] - mu * sc
    out_ref[...] = jnp.maximum(h2_ref[...] * sc + sh, 0.0)


def _k3(h2, s2, q2, g2, b2):
    full = lambda r, c: pl.BlockSpec((r, c), lambda i: (0, 0))
    return pl.pallas_call(
        _k3_body,
        grid=(TWO_E // TE,),
        in_specs=[pl.BlockSpec((TE, H), lambda i: (i, 0)),
                  full(1, H), full(1, H), full(1, H), full(1, H)],
        out_specs=pl.BlockSpec((TE, H), lambda i: (i, 0)),
        out_shape=jax.ShapeDtypeStruct((TWO_E, H), _F32),
    )(h2, s2, q2, g2, b2)


def _k4_body(nr_ref, es_ref, ns_ref, degn_ref,
             v0_ref, v1_ref, v23_ref, v3_ref,
             g1_ref, b1_ref, w2_ref, g2_ref, b2_ref, out_ref):
    deg = degn_ref[...]
    nr = nr_ref[...]
    d2nr = deg * deg * nr
    z1 = (_mm(nr, v0_ref[...]) + _mm(es_ref[...], v1_ref[...])
          + _mm(d2nr, v23_ref[...]) + _mm(ns_ref[...], v3_ref[...]))
    m1 = jnp.mean(z1, axis=0, keepdims=True)
    v1 = jnp.mean((z1 - m1) * (z1 - m1), axis=0, keepdims=True)
    x = jnp.maximum((z1 - m1) * lax.rsqrt(v1 + 1e-5) * g1_ref[...]
                    + b1_ref[...], 0.0)
    z2 = _mm(x, w2_ref[...])
    m2 = jnp.mean(z2, axis=0, keepdims=True)
    v2 = jnp.mean((z2 - m2) * (z2 - m2), axis=0, keepdims=True)
    out_ref[...] = jnp.maximum((z2 - m2) * lax.rsqrt(v2 + 1e-5) * g2_ref[...]
                               + b2_ref[...], 0.0)


def _k4(nr, es, ns, degn, v0, v1, v23, v3, g1, b1, w2, g2, b2):
    return pl.pallas_call(
        _k4_body,
        out_shape=jax.ShapeDtypeStruct((N, H), _F32),
    )(nr, es, ns, degn, v0, v1, v23, v3, g1, b1, w2, g2, b2)


def kernel(node_rep, edge_rep, edge_index,
           node_W1, node_g1, node_b1, node_W2, node_g2, node_b2,
           edge_W1, edge_g1, edge_b1, edge_W2, edge_g2, edge_b2):
    u = edge_index[0].astype(jnp.int32)
    v = edge_index[1].astype(jnp.int32)
    nid2 = jnp.stack([u, v], axis=1).reshape(TWO_E // 128, 128)
    pnid2 = jnp.stack([v, u], axis=1).reshape(TWO_E // 128, 128)

    c11 = _gram(edge_rep, edge_rep)
    p_rows, es, ns, deg128 = _sc_main(nid2, pnid2, edge_rep, node_rep)
    degn = deg128[:, 0:1]
    s_tbl = _ks(node_rep, degn)
    c13 = _gram(edge_rep, p_rows)
    sa_rows = _sc_aux(nid2, s_tbl)

    w1a = edge_W1[:H]
    w1bc = edge_W1[H:2 * H] + edge_W1[2 * H:]
    w1c = edge_W1[2 * H:]
    g1 = edge_g1.reshape(1, 2 * H)
    b1 = edge_b1.reshape(1, 2 * H)
    g2 = edge_g2.reshape(1, H)
    b2 = edge_b2.reshape(1, H)

    scale1, shift1 = _k1(c11, c13, es, ns, node_rep, degn,
                         w1a, w1bc, w1c, g1, b1)
    h2, s2, q2 = _k2(edge_rep, sa_rows, p_rows, scale1, shift1,
                     w1a, w1bc, w1c, edge_W2)
    edge_out = _k3(h2, s2, q2, g2, b2)

    nv0 = node_W1[:H]
    nv1 = node_W1[H:2 * H]
    nv23 = node_W1[2 * H:3 * H] + node_W1[3 * H:]
    nv3 = node_W1[3 * H:]
    node_out = _k4(node_rep, es, ns, degn, nv0, nv1, nv23, nv3,
                   node_g1.reshape(1, 2 * H), node_b1.reshape(1, 2 * H),
                   node_W2, node_g2.reshape(1, H), node_b2.reshape(1, H))
    return (node_out, edge_out)


# speedup vs baseline: 1.1884x; 1.1884x over previous
import functools

import jax
import jax.numpy as jnp
from jax import lax
from jax.experimental import pallas as pl
from jax.experimental.pallas import tpu as pltpu
from jax.experimental.pallas import tpu_sc as plsc

N = 10000
E = 320000
TWO_E = 2 * E
H = 128
SCH = 1024
NSC = TWO_E // SCH
TE = 1024

_BASE = 624


def _tile_start(s):
    return 8 * (78 * s + jnp.minimum(s, 2))


def _mesh():
    return plsc.VectorSubcoreMesh(
        core_axis_name="c", subcore_axis_name="s",
        num_cores=2, num_subcores=16)

_F32 = jnp.float32


def _mm(x, w):
    return lax.dot_general(x, w, (((1,), (0,)), ((), ())),
                           preferred_element_type=_F32)


def _mmT(x, y):
    return lax.dot_general(x, y, (((0,), (0,)), ((), ())),
                           preferred_element_type=_F32)


def _sc_main(nid2, pnid2, er, nr):
    out_type = (
        jax.ShapeDtypeStruct((TWO_E, H), _F32),
        jax.ShapeDtypeStruct((N, H), _F32),
        jax.ShapeDtypeStruct((N, H), _F32),
        jax.ShapeDtypeStruct((N, H), _F32),
    )
    scratch = [
        pltpu.VMEM((8, 128), jnp.int32),
        pltpu.VMEM((8, 128), jnp.int32),
        pltpu.VMEM((256, H), _F32),
        pltpu.VMEM_SHARED((N, H), _F32),
        pltpu.SemaphoreType.DMA,
    ]

    @functools.partial(pl.kernel, out_type=out_type, mesh=_mesh(),
                       scratch_types=scratch)
    def k(nid_h, pnid_h, er_h, nr_h, p_h, es_h, ns_h, deg_h,
          idx_v, pidx_v, buf_v, acc_sh, sem):
        c = lax.axis_index("c")
        s = lax.axis_index("s")
        zeros16 = jnp.zeros((16,), _F32)
        ones16 = jnp.ones((16,), _F32)
        start = _tile_start(s)
        nck = jnp.where(s < 1, NSC // 16 + 1, NSC // 16)

        def fill_buf(val16, nrows):
            def frow(r, _):
                for l in range(H // 16):
                    buf_v[r, pl.ds(l * 16, 16)] = val16
                return 0
            lax.fori_loop(0, nrows, frow, 0)

        def zero_acc():
            pltpu.sync_copy(buf_v, acc_sh.at[pl.ds(start, 256)])
            pltpu.sync_copy(buf_v, acc_sh.at[pl.ds(start + 256, 256)])
            pltpu.sync_copy(buf_v.at[pl.ds(0, _BASE - 512)],
                            acc_sh.at[pl.ds(start + 512, _BASE - 512)])

            @pl.when(s < 2)
            def _():
                pltpu.sync_copy(buf_v.at[pl.ds(0, 8)],
                                acc_sh.at[pl.ds(start + _BASE, 8)])

        def copy_acc_out(dst_h):
            pltpu.sync_copy(acc_sh.at[pl.ds(start, _BASE)],
                            dst_h.at[pl.ds(start, _BASE)])

            @pl.when(s < 2)
            def _():
                pltpu.sync_copy(acc_sh.at[pl.ds(start + _BASE, 8)],
                                dst_h.at[pl.ds(start + _BASE, 8)])

        fill_buf(zeros16, 256)
        zero_acc()

        @pl.when(c == 0)
        def _():
            plsc.subcore_barrier()
            fill_buf(ones16, 128)

            def dchunk(kk, _):
                sc = s + kk * 16
                pltpu.sync_copy(nid_h.at[pl.ds(sc * 8, 8)], idx_v)
                for j in range(8):
                    pltpu.sync_copy(buf_v.at[pl.ds(0, 128)],
                                    acc_sh.at[idx_v.at[j]], add=True)
                return 0

            lax.fori_loop(0, nck, dchunk, 0)
            plsc.subcore_barrier()
            copy_acc_out(deg_h)
            fill_buf(zeros16, 128)
            plsc.subcore_barrier()
            zero_acc()

        plsc.subcore_barrier()

        def chunk(kk, _):
            sc = s + kk * 16
            g0 = sc * 8
            pltpu.sync_copy(nid_h.at[pl.ds(g0, 8)], idx_v)

            @pl.when(c == 0)
            def _():
                for q in range(4):
                    row0 = sc * SCH + q * 256
                    pltpu.sync_copy(er_h.at[pl.ds(row0, 256)], buf_v)
                    for j in range(2):
                        pltpu.sync_copy(
                            buf_v.at[pl.ds(j * 128, 128)],
                            acc_sh.at[idx_v.at[q * 2 + j]], add=True)

            @pl.when(c == 1)
            def _():
                pltpu.sync_copy(pnid_h.at[pl.ds(g0, 8)], pidx_v)
                for q in range(4):
                    row0 = sc * SCH + q * 256
                    d0 = pltpu.async_copy(nr_h.at[pidx_v.at[q * 2]],
                                          buf_v.at[pl.ds(0, 128)], sem)
                    d1 = pltpu.async_copy(nr_h.at[pidx_v.at[q * 2 + 1]],
                                          buf_v.at[pl.ds(128, 128)], sem)
                    d0.wait()
                    d1.wait()
                    pltpu.sync_copy(buf_v, p_h.at[pl.ds(row0, 256)])
                    for j in range(2):
                        pltpu.sync_copy(
                            buf_v.at[pl.ds(j * 128, 128)],
                            acc_sh.at[idx_v.at[q * 2 + j]], add=True)
            return 0

        lax.fori_loop(0, nck, chunk, 0)
        plsc.subcore_barrier()

        @pl.when(c == 0)
        def _():
            copy_acc_out(es_h)

        @pl.when(c == 1)
        def _():
            copy_acc_out(ns_h)

    return k(nid2, pnid2, er, nr)


def _sc_aux(nid2, s_tbl):
    out_type = jax.ShapeDtypeStruct((TWO_E, H), _F32)
    scratch = [
        pltpu.VMEM((8, 128), jnp.int32),
        pltpu.VMEM((256, H), _F32),
        pltpu.SemaphoreType.DMA,
    ]

    @functools.partial(pl.kernel, out_type=out_type, mesh=_mesh(),
                       scratch_types=scratch)
    def k(nid_h, s_h, sa_h, idx_v, ebuf_v, sem):
        c = lax.axis_index("c")
        s = lax.axis_index("s")
        wid = c * 16 + s

        nck = jnp.where(wid < NSC - 32 * (NSC // 32), NSC // 32 + 1,
                        NSC // 32)

        def chunk(kk, _):
            sc = wid + kk * 32
            g0 = sc * 8
            pltpu.sync_copy(nid_h.at[pl.ds(g0, 8)], idx_v)
            for q in range(4):
                row0 = sc * SCH + q * 256
                d0 = pltpu.async_copy(s_h.at[idx_v.at[q * 2]],
                                      ebuf_v.at[pl.ds(0, 128)], sem)
                d1 = pltpu.async_copy(s_h.at[idx_v.at[q * 2 + 1]],
                                      ebuf_v.at[pl.ds(128, 128)], sem)
                d0.wait()
                d1.wait()
                pltpu.sync_copy(ebuf_v, sa_h.at[pl.ds(row0, 256)])
            return 0

        lax.fori_loop(0, nck, chunk, 0)

    return k(nid2, s_tbl)


def _ks_body(nr_ref, degn_ref, s_ref):
    s_ref[...] = degn_ref[...] * nr_ref[...]


def _ks(nr, degn):
    return pl.pallas_call(
        _ks_body,
        out_shape=jax.ShapeDtypeStruct((N, H), _F32),
    )(nr, degn)


def _k0_body(er_ref, p_ref, c11_ref, c13_ref):
    i = pl.program_id(0)
    er = er_ref[...]
    c11 = _mmT(er, er)
    c13 = _mmT(er, p_ref[...])

    @pl.when(i == 0)
    def _():
        c11_ref[...] = c11
        c13_ref[...] = c13

    @pl.when(i != 0)
    def _():
        c11_ref[...] += c11
        c13_ref[...] += c13


def _k0(er, p):
    return pl.pallas_call(
        _k0_body,
        grid=(TWO_E // TE,),
        in_specs=[pl.BlockSpec((TE, H), lambda i: (i, 0)),
                  pl.BlockSpec((TE, H), lambda i: (i, 0))],
        out_specs=[pl.BlockSpec((H, H), lambda i: (0, 0)),
                   pl.BlockSpec((H, H), lambda i: (0, 0))],
        out_shape=[jax.ShapeDtypeStruct((H, H), _F32),
                   jax.ShapeDtypeStruct((H, H), _F32)],
    )(er, p)


def _k1_body(c11_ref, c13_ref, es_ref, ns_ref, nr_ref, degn_ref,
             wa_ref, wb_ref, wc_ref, g1_ref, b1_ref,
             scale1_ref, shift1_ref):
    deg = degn_ref[...]
    nr = nr_ref[...]
    es = es_ref[...]
    ns = ns_ref[...]
    d1nr = deg * nr
    d2nr = deg * d1nr
    n = jnp.float32(TWO_E)

    wa = wa_ref[...]
    wb = wb_ref[...]
    wc = wc_ref[...]

    ey_es = jnp.sum(es, axis=0, keepdims=True)
    ey_d2 = jnp.sum(d2nr, axis=0, keepdims=True)
    ey_d1 = jnp.sum(d1nr, axis=0, keepdims=True)
    mu1 = (_mm(ey_es, wa) + _mm(ey_d2, wb) + _mm(ey_d1, wc)) / n

    u11 = c11_ref[...]
    u13 = c13_ref[...]
    u12 = _mmT(deg * es, nr)
    u22 = _mmT(deg * d2nr, nr)
    u23 = _mmT(d1nr, ns)
    u33 = _mmT(d1nr, nr)

    mt = _mm(u11, wa) + _mm(u12, wb) + _mm(u13, wc)
    mm_ = _mmT(u12, wa) + _mm(u22, wb) + _mm(u23, wc)
    mb = _mmT(u13, wa) + _mmT(u23, wb) + _mm(u33, wc)
    eh2 = (jnp.sum(wa * mt, axis=0, keepdims=True)
           + jnp.sum(wb * mm_, axis=0, keepdims=True)
           + jnp.sum(wc * mb, axis=0, keepdims=True)) / n

    var1 = eh2 - mu1 * mu1
    isig = lax.rsqrt(var1 + 1e-5)
    g1 = g1_ref[...]
    scale1_ref[...] = isig * g1
    shift1_ref[...] = b1_ref[...] - mu1 * isig * g1


def _k1(c11, c13, es, ns, nr, degn, wa, wb, wc, g1, b1):
    return pl.pallas_call(
        _k1_body,
        out_shape=[jax.ShapeDtypeStruct((1, 2 * H), _F32),
                   jax.ShapeDtypeStruct((1, 2 * H), _F32)],
    )(c11, c13, es, ns, nr, degn, wa, wb, wc, g1, b1)


def _k2_body(er_ref, sa_ref, p_ref, sc1_ref, sh1_ref,
             wa_ref, wb_ref, wc_ref, w2_ref,
             h2_ref, s2_ref, q2_ref):
    i = pl.program_id(0)
    h1 = (_mm(er_ref[...], wa_ref[...]) + _mm(sa_ref[...], wb_ref[...])
          + _mm(p_ref[...], wc_ref[...]))
    x = jnp.maximum(h1 * sc1_ref[...] + sh1_ref[...], 0.0)
    h2 = _mm(x, w2_ref[...])
    h2_ref[...] = h2.astype(jnp.bfloat16)

    se = jnp.sum(h2, axis=0, keepdims=True)
    qe = jnp.sum(h2 * h2, axis=0, keepdims=True)

    @pl.when(i == 0)
    def _():
        s2_ref[...] = se
        q2_ref[...] = qe

    @pl.when(i != 0)
    def _():
        s2_ref[...] += se
        q2_ref[...] += qe


def _k2(er, sa, p, scale1, shift1, wa, wb, wc, w2):
    full = lambda r, c: pl.BlockSpec((r, c), lambda i: (0, 0))
    row = lambda: pl.BlockSpec((TE, H), lambda i: (i, 0))
    return pl.pallas_call(
        _k2_body,
        grid=(TWO_E // TE,),
        in_specs=[row(), row(), row(),
                  full(1, 2 * H), full(1, 2 * H),
                  full(H, 2 * H), full(H, 2 * H), full(H, 2 * H),
                  full(2 * H, H)],
        out_specs=[pl.BlockSpec((TE, H), lambda i: (i, 0)),
                   pl.BlockSpec((1, H), lambda i: (0, 0)),
                   pl.BlockSpec((1, H), lambda i: (0, 0))],
        out_shape=[jax.ShapeDtypeStruct((TWO_E, H), jnp.bfloat16),
                   jax.ShapeDtypeStruct((1, H), _F32),
                   jax.ShapeDtypeStruct((1, H), _F32)],
    )(er, sa, p, scale1, shift1, wa, wb, wc, w2)


def _k3_body(h2_ref, s2_ref, q2_ref, g2_ref, b2_ref, out_ref):
    n = jnp.float32(TWO_E)
    mu = s2_ref[...] / n
    var = q2_ref[...] / n - mu * mu
    sc = lax.rsqrt(var + 1e-5) * g2_ref[...]
    sh = b2_ref[...] - mu * sc
    out_ref[...] = jnp.maximum(h2_ref[...].astype(_F32) * sc + sh, 0.0)


def _k3(h2, s2, q2, g2, b2):
    full = lambda r, c: pl.BlockSpec((r, c), lambda i: (0, 0))
    return pl.pallas_call(
        _k3_body,
        grid=(TWO_E // TE,),
        in_specs=[pl.BlockSpec((TE, H), lambda i: (i, 0)),
                  full(1, H), full(1, H), full(1, H), full(1, H)],
        out_specs=pl.BlockSpec((TE, H), lambda i: (i, 0)),
        out_shape=jax.ShapeDtypeStruct((TWO_E, H), _F32),
    )(h2, s2, q2, g2, b2)


def _k4_body(nr_ref, es_ref, ns_ref, degn_ref,
             v0_ref, v1_ref, v23_ref, v3_ref,
             g1_ref, b1_ref, w2_ref, g2_ref, b2_ref, out_ref):
    deg = degn_ref[...]
    nr = nr_ref[...]
    d2nr = deg * deg * nr
    z1 = (_mm(nr, v0_ref[...]) + _mm(es_ref[...], v1_ref[...])
          + _mm(d2nr, v23_ref[...]) + _mm(ns_ref[...], v3_ref[...]))
    m1 = jnp.mean(z1, axis=0, keepdims=True)
    v1 = jnp.mean((z1 - m1) * (z1 - m1), axis=0, keepdims=True)
    x = jnp.maximum((z1 - m1) * lax.rsqrt(v1 + 1e-5) * g1_ref[...]
                    + b1_ref[...], 0.0)
    z2 = _mm(x, w2_ref[...])
    m2 = jnp.mean(z2, axis=0, keepdims=True)
    v2 = jnp.mean((z2 - m2) * (z2 - m2), axis=0, keepdims=True)
    out_ref[...] = jnp.maximum((z2 - m2) * lax.rsqrt(v2 + 1e-5) * g2_ref[...]
                               + b2_ref[...], 0.0)


def _k4(nr, es, ns, degn, v0, v1, v23, v3, g1, b1, w2, g2, b2):
    return pl.pallas_call(
        _k4_body,
        out_shape=jax.ShapeDtypeStruct((N, H), _F32),
    )(nr, es, ns, degn, v0, v1, v23, v3, g1, b1, w2, g2, b2)


def kernel(node_rep, edge_rep, edge_index,
           node_W1, node_g1, node_b1, node_W2, node_g2, node_b2,
           edge_W1, edge_g1, edge_b1, edge_W2, edge_g2, edge_b2):
    u = edge_index[0].astype(jnp.int32)
    v = edge_index[1].astype(jnp.int32)
    nid2 = jnp.stack([u, v], axis=1).reshape(TWO_E // 128, 128)
    pnid2 = jnp.stack([v, u], axis=1).reshape(TWO_E // 128, 128)

    p_rows, es, ns, deg128 = _sc_main(nid2, pnid2, edge_rep, node_rep)
    degn = deg128[:, 0:1]
    s_tbl = _ks(node_rep, degn)
    sa_rows = _sc_aux(nid2, s_tbl)
    c11, c13 = _k0(edge_rep, p_rows)

    w1a = edge_W1[:H]
    w1bc = edge_W1[H:2 * H] + edge_W1[2 * H:]
    w1c = edge_W1[2 * H:]
    g1 = edge_g1.reshape(1, 2 * H)
    b1 = edge_b1.reshape(1, 2 * H)
    g2 = edge_g2.reshape(1, H)
    b2 = edge_b2.reshape(1, H)

    scale1, shift1 = _k1(c11, c13, es, ns, node_rep, degn,
                         w1a, w1bc, w1c, g1, b1)
    h2, s2, q2 = _k2(edge_rep, sa_rows, p_rows, scale1, shift1,
                     w1a, w1bc, w1c, edge_W2)
    edge_out = _k3(h2, s2, q2, g2, b2)

    nv0 = node_W1[:H]
    nv1 = node_W1[H:2 * H]
    nv23 = node_W1[2 * H:3 * H] + node_W1[3 * H:]
    nv3 = node_W1[3 * H:]
    node_out = _k4(node_rep, es, ns, degn, nv0, nv1, nv23, nv3,
                   node_g1.reshape(1, 2 * H), node_b1.reshape(1, 2 * H),
                   node_W2, node_g2.reshape(1, H), node_b2.reshape(1, H))
    return (node_out, edge_out)
